# Initial kernel scaffold; baseline (speedup 1.0000x reference)
import jax
import jax.numpy as jnp
from jax import lax
from jax.experimental import pallas as pl
from jax.experimental.pallas import tpu as pltpu
from jax.experimental.pallas import tpu_sc as plsc

N = 100000
S = 4
D = 128
B = 512
BLK = 128
NFULL = N // BLK
REM = N - NFULL * BLK
NW = 32


def _sc_body(nd_ref, idx_ref, parts_ref, cnts_ref,
             buf, idxbuf, onesbuf, zbuf, bufr, idxr, acc_sh, cnt_sh):
    cid = lax.axis_index("c")
    sid = lax.axis_index("s")
    wid = sid * 2 + cid

    zeros16 = jnp.zeros((16,), jnp.float32)
    ones16 = jnp.ones((16,), jnp.float32)

    def zrow(i, carry):
        for s in range(S):
            for j in range(D // 16):
                bufr[i, s, pl.ds(j * 16, 16)] = zeros16
        zbuf[i, :] = zeros16
        return carry
    lax.fori_loop(0, 32, zrow, 0)

    def orow(i, carry):
        onesbuf[i, :] = ones16
        return carry
    lax.fori_loop(0, BLK, orow, 0)

    acc_rows = pl.ds(sid * 32, 32)
    pltpu.sync_copy(bufr, acc_sh.at[acc_rows])
    pltpu.sync_copy(zbuf, cnt_sh.at[acc_rows])
    plsc.subcore_barrier()

    nblk = (NFULL - 1 - wid) // NW + 1

    def step(i, carry):
        blk = wid + i * NW
        base = pl.multiple_of(blk * BLK, BLK)
        pltpu.sync_copy(nd_ref.at[pl.ds(base, BLK)], buf)
        pltpu.sync_copy(idx_ref.at[pl.ds(base, BLK)], idxbuf)
        pltpu.sync_copy(buf, acc_sh.at[idxbuf], add=True)
        pltpu.sync_copy(onesbuf, cnt_sh.at[idxbuf], add=True)
        return carry
    lax.fori_loop(0, nblk, step, 0)

    @pl.when(jnp.logical_and(cid == 0, sid == 0))
    def _():
        base = pl.multiple_of(NFULL * BLK, 32)
        pltpu.sync_copy(nd_ref.at[pl.ds(base, REM)], bufr)
        pltpu.sync_copy(idx_ref.at[pl.ds(base, REM)], idxr)
        pltpu.sync_copy(bufr, acc_sh.at[idxr], add=True)
        pltpu.sync_copy(onesbuf.at[pl.ds(0, REM)], cnt_sh.at[idxr], add=True)

    plsc.subcore_barrier()

    pltpu.sync_copy(acc_sh.at[acc_rows], parts_ref.at[cid, acc_rows])
    pltpu.sync_copy(cnt_sh.at[acc_rows], cnts_ref.at[cid, acc_rows])


def _sc_accumulate(nd, idx):
    run = pl.kernel(
        _sc_body,
        out_type=(
            jax.ShapeDtypeStruct((2, B, S, D), jnp.float32),
            jax.ShapeDtypeStruct((2, B, 16), jnp.float32),
        ),
        mesh=plsc.VectorSubcoreMesh(core_axis_name="c", subcore_axis_name="s"),
        scratch_types=[
            pltpu.VMEM((BLK, S, D), jnp.float32),
            pltpu.VMEM((BLK,), jnp.int32),
            pltpu.VMEM((BLK, 16), jnp.float32),
            pltpu.VMEM((32, 16), jnp.float32),
            pltpu.VMEM((32, S, D), jnp.float32),
            pltpu.VMEM((32,), jnp.int32),
            pltpu.VMEM_SHARED((B, S, D), jnp.float32),
            pltpu.VMEM_SHARED((B, 16), jnp.float32),
        ],
    )
    return run(nd, idx)


def _finalize(parts, cnts):
    def body(parts_ref, cnts_ref, out_ref):
        p = parts_ref[...]
        q = p[0] + p[1]
        acc = q[:, 0, :] + q[:, 1, :] + q[:, 2, :] + q[:, 3, :]
        c = cnts_ref[...]
        cnt = c[0, :, 0:1] + c[1, :, 0:1]
        out_ref[...] = acc * (0.25 / jnp.maximum(cnt, 1.0))

    return pl.pallas_call(
        body,
        out_shape=jax.ShapeDtypeStruct((B, D), jnp.float32),
    )(parts, cnts)


def kernel(node_distributions, batch_idx):
    idx32 = batch_idx.astype(jnp.int32)
    parts, cnts = _sc_accumulate(node_distributions, idx32)
    return _finalize(parts, cnts)


# speedup vs baseline: 4.3919x; 4.3919x over previous
import jax
import jax.numpy as jnp
from jax import lax
from jax.experimental import pallas as pl
from jax.experimental.pallas import tpu as pltpu
from jax.experimental.pallas import tpu_sc as plsc

N = 100000
S = 4
D = 128
B = 512
BLK = 128
NFULL = N // BLK
REM = N - NFULL * BLK
NW = 32


def _sc_body(nd_ref, idx_ref, parts_ref, cnts_ref,
             buf, idxbuf, onesbuf, zbuf, bufr, idxr, acc_sh, cnt_sh):
    cid = lax.axis_index("c")
    sid = lax.axis_index("s")
    wid = sid * 2 + cid

    zeros16 = jnp.zeros((16,), jnp.float32)
    ones16 = jnp.ones((16,), jnp.float32)

    def zrow(i, carry):
        for s in range(S):
            for j in range(D // 16):
                bufr[i, s, pl.ds(j * 16, 16)] = zeros16
        for j in range(D // 16):
            zbuf[i, pl.ds(j * 16, 16)] = zeros16
        return carry
    lax.fori_loop(0, 32, zrow, 0)

    def orow(i, carry):
        for j in range(D // 16):
            onesbuf[i, pl.ds(j * 16, 16)] = ones16
        return carry
    lax.fori_loop(0, BLK, orow, 0)

    acc_rows = pl.ds(sid * 32, 32)
    pltpu.sync_copy(bufr, acc_sh.at[acc_rows])
    pltpu.sync_copy(zbuf, cnt_sh.at[acc_rows])
    plsc.subcore_barrier()

    nblk = (NFULL - 1 - wid) // NW + 1

    def step(i, carry):
        blk = wid + i * NW
        base = pl.multiple_of(blk * BLK, BLK)
        pltpu.sync_copy(nd_ref.at[pl.ds(base, BLK)], buf)
        pltpu.sync_copy(idx_ref.at[pl.ds(base, BLK)], idxbuf)
        pltpu.sync_copy(buf, acc_sh.at[idxbuf], add=True)
        pltpu.sync_copy(onesbuf, cnt_sh.at[idxbuf], add=True)
        return carry
    lax.fori_loop(0, nblk, step, 0)

    @pl.when(jnp.logical_and(cid == 0, sid == 0))
    def _():
        base = pl.multiple_of(NFULL * BLK, 32)
        pltpu.sync_copy(nd_ref.at[pl.ds(base, REM)], bufr)
        pltpu.sync_copy(idx_ref.at[pl.ds(base, REM)], idxr)
        pltpu.sync_copy(bufr, acc_sh.at[idxr], add=True)
        pltpu.sync_copy(onesbuf.at[pl.ds(0, REM)], cnt_sh.at[idxr], add=True)

    plsc.subcore_barrier()

    pltpu.sync_copy(acc_sh.at[acc_rows], parts_ref.at[cid, acc_rows])
    pltpu.sync_copy(cnt_sh.at[acc_rows], cnts_ref.at[cid, acc_rows])


def _sc_accumulate(nd, idx):
    run = pl.kernel(
        _sc_body,
        out_type=(
            jax.ShapeDtypeStruct((2, B, S, D), jnp.float32),
            jax.ShapeDtypeStruct((2, B, D), jnp.float32),
        ),
        mesh=plsc.VectorSubcoreMesh(core_axis_name="c", subcore_axis_name="s"),
        scratch_types=[
            pltpu.VMEM((BLK, S, D), jnp.float32),
            pltpu.VMEM((BLK,), jnp.int32),
            pltpu.VMEM((BLK, D), jnp.float32),
            pltpu.VMEM((32, D), jnp.float32),
            pltpu.VMEM((32, S, D), jnp.float32),
            pltpu.VMEM((32,), jnp.int32),
            pltpu.VMEM_SHARED((B, S, D), jnp.float32),
            pltpu.VMEM_SHARED((B, D), jnp.float32),
        ],
    )
    return run(nd, idx)


def _finalize(parts, cnts):
    def body(parts_ref, cnts_ref, out_ref):
        p = parts_ref[...]
        q = p[0] + p[1]
        acc = q[:, 0, :] + q[:, 1, :] + q[:, 2, :] + q[:, 3, :]
        c = cnts_ref[...]
        cnt = c[0, :, 0:1] + c[1, :, 0:1]
        out_ref[...] = acc * (0.25 / jnp.maximum(cnt, 1.0))

    return pl.pallas_call(
        body,
        out_shape=jax.ShapeDtypeStruct((B, D), jnp.float32),
    )(parts, cnts)


def kernel(node_distributions, batch_idx):
    idx32 = batch_idx.astype(jnp.int32)
    parts, cnts = _sc_accumulate(node_distributions, idx32)
    return _finalize(parts, cnts)


# speedup vs baseline: 6.3118x; 1.4371x over previous
import jax
import jax.numpy as jnp
from jax import lax
from jax.experimental import pallas as pl
from jax.experimental.pallas import tpu as pltpu
from jax.experimental.pallas import tpu_sc as plsc

N = 100000
S = 4
D = 128
B = 512
BLK = 64
NFULL = N // BLK
REM = N - NFULL * BLK
NW = 32


def _sc_body(nd_ref, idx_ref, parts_ref, cnts_ref,
             buf0, buf1, idx0, idx1, onesbuf, zbuf, idxr,
             sem0, sem1, acc_sh, cnt_sh):
    cid = lax.axis_index("c")
    sid = lax.axis_index("s")
    wid = sid * 2 + cid

    zeros16 = jnp.zeros((16,), jnp.float32)
    ones16 = jnp.ones((16,), jnp.float32)

    def zrow(i, carry):
        for s in range(S):
            for j in range(D // 16):
                buf0[i, s, pl.ds(j * 16, 16)] = zeros16
        for j in range(D // 16):
            zbuf[i, pl.ds(j * 16, 16)] = zeros16
        return carry
    lax.fori_loop(0, 32, zrow, 0)

    def orow(i, carry):
        for j in range(D // 16):
            onesbuf[i, pl.ds(j * 16, 16)] = ones16
        return carry
    lax.fori_loop(0, BLK, orow, 0)

    acc_rows = pl.ds(sid * 32, 32)
    pltpu.sync_copy(buf0.at[pl.ds(0, 32)], acc_sh.at[acc_rows])
    pltpu.sync_copy(zbuf, cnt_sh.at[acc_rows])
    plsc.subcore_barrier()

    nblk = (NFULL - 1 - wid) // NW + 1
    bufs = (buf0, buf1)
    idxs = (idx0, idx1)
    sems = (sem0, sem1)

    def issue(slot, i):
        blk = wid + i * NW
        base = pl.multiple_of(blk * BLK, BLK)
        pltpu.async_copy(nd_ref.at[pl.ds(base, BLK)], bufs[slot], sems[slot])
        pltpu.async_copy(idx_ref.at[pl.ds(base, BLK)], idxs[slot], sems[slot])

    def drain_and_scatter(slot):
        pltpu.make_async_copy(nd_ref.at[pl.ds(0, BLK)], bufs[slot],
                              sems[slot]).wait()
        pltpu.make_async_copy(idx_ref.at[pl.ds(0, BLK)], idxs[slot],
                              sems[slot]).wait()
        pltpu.sync_copy(bufs[slot], acc_sh.at[idxs[slot]], add=True)
        pltpu.sync_copy(onesbuf, cnt_sh.at[idxs[slot]], add=True)

    issue(0, 0)

    def step(i, carry):
        for slot in (0, 1):
            @pl.when((i & 1) == slot)
            def _():
                @pl.when(i + 1 < nblk)
                def _():
                    issue(1 - slot, i + 1)
                drain_and_scatter(slot)
        return carry
    lax.fori_loop(0, nblk, step, 0)

    @pl.when(jnp.logical_and(cid == 0, sid == 0))
    def _():
        base = pl.multiple_of(NFULL * BLK, 32)
        pltpu.sync_copy(nd_ref.at[pl.ds(base, REM)], buf0.at[pl.ds(0, REM)])
        pltpu.sync_copy(idx_ref.at[pl.ds(base, REM)], idxr)
        pltpu.sync_copy(buf0.at[pl.ds(0, REM)], acc_sh.at[idxr], add=True)
        pltpu.sync_copy(onesbuf.at[pl.ds(0, REM)], cnt_sh.at[idxr], add=True)

    plsc.subcore_barrier()

    pltpu.sync_copy(acc_sh.at[acc_rows], parts_ref.at[cid, acc_rows])
    pltpu.sync_copy(cnt_sh.at[acc_rows], cnts_ref.at[cid, acc_rows])


def _sc_accumulate(nd, idx):
    run = pl.kernel(
        _sc_body,
        out_type=(
            jax.ShapeDtypeStruct((2, B, S, D), jnp.float32),
            jax.ShapeDtypeStruct((2, B, D), jnp.float32),
        ),
        mesh=plsc.VectorSubcoreMesh(core_axis_name="c", subcore_axis_name="s"),
        scratch_types=[
            pltpu.VMEM((BLK, S, D), jnp.float32),
            pltpu.VMEM((BLK, S, D), jnp.float32),
            pltpu.VMEM((BLK,), jnp.int32),
            pltpu.VMEM((BLK,), jnp.int32),
            pltpu.VMEM((BLK, D), jnp.float32),
            pltpu.VMEM((32, D), jnp.float32),
            pltpu.VMEM((32,), jnp.int32),
            pltpu.SemaphoreType.DMA,
            pltpu.SemaphoreType.DMA,
            pltpu.VMEM_SHARED((B, S, D), jnp.float32),
            pltpu.VMEM_SHARED((B, D), jnp.float32),
        ],
    )
    return run(nd, idx)


def _finalize(parts, cnts):
    def body(parts_ref, cnts_ref, out_ref):
        p = parts_ref[...]
        q = p[0] + p[1]
        acc = q[:, 0, :] + q[:, 1, :] + q[:, 2, :] + q[:, 3, :]
        c = cnts_ref[...]
        cnt = c[0, :, 0:1] + c[1, :, 0:1]
        out_ref[...] = acc * (0.25 / jnp.maximum(cnt, 1.0))

    return pl.pallas_call(
        body,
        out_shape=jax.ShapeDtypeStruct((B, D), jnp.float32),
    )(parts, cnts)


def kernel(node_distributions, batch_idx):
    idx32 = batch_idx.astype(jnp.int32)
    parts, cnts = _sc_accumulate(node_distributions, idx32)
    return _finalize(parts, cnts)


# speedup vs baseline: 6.7940x; 1.0764x over previous
import jax
import jax.numpy as jnp
from jax import lax
from jax.experimental import pallas as pl
from jax.experimental.pallas import tpu as pltpu
from jax.experimental.pallas import tpu_sc as plsc

N = 100000
S = 4
D = 128
B = 512
BLK = 64
NFULL = N // BLK
REM = N - NFULL * BLK
NW = 32
SCAN = 3136
SCAN_LAST = N - (NW - 1) * SCAN


def _sc_body(nd_ref, idx_ref, parts_ref, firsts_ref,
             buf0, buf1, idx0, idx1, idxall, first, idxr,
             sem0, sem1, acc_sh):
    cid = lax.axis_index("c")
    sid = lax.axis_index("s")
    wid = sid * 2 + cid

    zeros16 = jnp.zeros((16,), jnp.float32)
    sent16 = jnp.full((16,), N, jnp.int32)

    def zrow(i, carry):
        for s in range(S):
            for j in range(D // 16):
                buf0[i, s, pl.ds(j * 16, 16)] = zeros16
        first[pl.ds(i * 16, 16)] = sent16
        return carry
    lax.fori_loop(0, 32, zrow, 0)

    acc_rows = pl.ds(sid * 32, 32)
    pltpu.sync_copy(buf0.at[pl.ds(0, 32)], acc_sh.at[acc_rows])
    plsc.subcore_barrier()

    start = wid * SCAN
    idxall[pl.ds(0, 16)] = jnp.full((16,), -1, jnp.int32)

    @pl.when(wid > 0)
    def _():
        pltpu.sync_copy(idx_ref.at[pl.ds(start - 8, 8)], idxall.at[pl.ds(8, 8)])

    @pl.when(wid < NW - 1)
    def _():
        pltpu.sync_copy(idx_ref.at[pl.ds(start, SCAN)],
                        idxall.at[pl.ds(16, SCAN)])

    @pl.when(wid == NW - 1)
    def _():
        pltpu.sync_copy(idx_ref.at[pl.ds(start, SCAN_LAST)],
                        idxall.at[pl.ds(16, SCAN_LAST)])

    nsteps = jnp.where(wid < NW - 1, SCAN // 16, SCAN_LAST // 16)
    lane = lax.iota(jnp.int32, 16)

    _dnums = lax.GatherDimensionNumbers(
        offset_dims=(), collapsed_slice_dims=(0,), start_index_map=(0,))

    def vgather(x, ix):
        return lax.gather(x, ix[:, None], _dnums, (1,),
                          mode=lax.GatherScatterMode.PROMISE_IN_BOUNDS)

    head = idxall[pl.ds(0, 16)]
    last_ix = jnp.full((16,), 15, jnp.int32)
    carry0 = vgather(head, last_ix)
    shift_ix = jnp.maximum(lane - 1, 0)

    def scan_step(t, carry):
        cur = idxall[pl.ds(16 + t * 16, 16)]
        prev_in = vgather(cur, shift_ix)
        prev = jnp.where(lane == 0, carry, prev_in)
        m = cur != prev
        pos = start + t * 16 + lane
        plsc.store_scatter(first, [cur], pos, mask=m)
        return vgather(cur, last_ix)
    lax.fori_loop(0, nsteps, scan_step, carry0)

    nblk = (NFULL - 1 - wid) // NW + 1
    bufs = (buf0, buf1)
    idxs = (idx0, idx1)
    sems = (sem0, sem1)

    def issue(slot, i):
        blk = wid + i * NW
        base = pl.multiple_of(blk * BLK, BLK)
        pltpu.async_copy(nd_ref.at[pl.ds(base, BLK)], bufs[slot], sems[slot])
        pltpu.async_copy(idx_ref.at[pl.ds(base, BLK)], idxs[slot], sems[slot])

    def drain_and_scatter(slot):
        pltpu.make_async_copy(nd_ref.at[pl.ds(0, BLK)], bufs[slot],
                              sems[slot]).wait()
        pltpu.make_async_copy(idx_ref.at[pl.ds(0, BLK)], idxs[slot],
                              sems[slot]).wait()
        pltpu.sync_copy(bufs[slot], acc_sh.at[idxs[slot]], add=True)

    issue(0, 0)

    def step(i, carry):
        for slot in (0, 1):
            @pl.when((i & 1) == slot)
            def _():
                @pl.when(i + 1 < nblk)
                def _():
                    issue(1 - slot, i + 1)
                drain_and_scatter(slot)
        return carry
    lax.fori_loop(0, nblk, step, 0)

    @pl.when(jnp.logical_and(cid == 0, sid == 0))
    def _():
        base = pl.multiple_of(NFULL * BLK, 32)
        pltpu.sync_copy(nd_ref.at[pl.ds(base, REM)], buf0.at[pl.ds(0, REM)])
        pltpu.sync_copy(idx_ref.at[pl.ds(base, REM)], idxr)
        pltpu.sync_copy(buf0.at[pl.ds(0, REM)], acc_sh.at[idxr], add=True)

    plsc.subcore_barrier()

    pltpu.sync_copy(acc_sh.at[acc_rows], parts_ref.at[cid, acc_rows])
    pltpu.sync_copy(first, firsts_ref.at[cid, sid])


def _sc_accumulate(nd, idx):
    run = pl.kernel(
        _sc_body,
        out_type=(
            jax.ShapeDtypeStruct((2, B, S, D), jnp.float32),
            jax.ShapeDtypeStruct((2, 16, B), jnp.int32),
        ),
        mesh=plsc.VectorSubcoreMesh(core_axis_name="c", subcore_axis_name="s"),
        compiler_params=pltpu.CompilerParams(needs_layout_passes=False),
        scratch_types=[
            pltpu.VMEM((BLK, S, D), jnp.float32),
            pltpu.VMEM((BLK, S, D), jnp.float32),
            pltpu.VMEM((BLK,), jnp.int32),
            pltpu.VMEM((BLK,), jnp.int32),
            pltpu.VMEM((16 + SCAN,), jnp.int32),
            pltpu.VMEM((B,), jnp.int32),
            pltpu.VMEM((32,), jnp.int32),
            pltpu.SemaphoreType.DMA,
            pltpu.SemaphoreType.DMA,
            pltpu.VMEM_SHARED((B, S, D), jnp.float32),
        ],
    )
    return run(nd, idx)


def _finalize(parts, firsts):
    def body(parts_ref, firsts_ref, out_ref):
        p = parts_ref[...]
        q = p[0] + p[1]
        acc = q[:, 0, :] + q[:, 1, :] + q[:, 2, :] + q[:, 3, :]

        f = firsts_ref[...].reshape(NW, B)
        F = jnp.min(f, axis=0).reshape(B, 1)
        k = 1
        while k < B:
            shifted = jnp.concatenate(
                [F[k:, :], jnp.full((k, 1), N, jnp.int32)], axis=0)
            F = jnp.minimum(F, shifted)
            k *= 2
        Fnext = jnp.concatenate(
            [F[1:, :], jnp.full((1, 1), N, jnp.int32)], axis=0)
        cnt = (Fnext - F).astype(jnp.float32)
        out_ref[...] = acc * (0.25 / jnp.maximum(cnt, 1.0))

    return pl.pallas_call(
        body,
        out_shape=jax.ShapeDtypeStruct((B, D), jnp.float32),
    )(parts, firsts)


def kernel(node_distributions, batch_idx):
    idx32 = batch_idx.astype(jnp.int32)
    parts, firsts = _sc_accumulate(node_distributions, idx32)
    return _finalize(parts, firsts)
